# baseline (speedup 1.0000x reference)
import jax
import jax.numpy as jnp
from jax import lax
from jax.experimental import pallas as pl
from jax.experimental.pallas import tpu as pltpu
from jax.experimental.pallas import tpu_sc as plsc

EMB = 128
OUT_W = 2 * EMB
ROWS = 16384
NUM_CORES = 2
NUM_SUBCORES = 16
NUM_WORKERS = NUM_CORES * NUM_SUBCORES
ROWS_PER_WORKER = ROWS // NUM_WORKERS
CHUNK = 128
CHUNKS_PER_WORKER = ROWS_PER_WORKER // CHUNK


def _bucket(lv):
    return ((jnp.minimum(lv, 2) - 1)
            + (jnp.minimum(lv, 4) >> 2)
            + (jnp.minimum(lv, 8) >> 3))


def _encode_body(len_hbm, tab_hbm, out_hbm,
                 len_v, idx_v, buf0, buf1, gsem0, gsem1, wsem0, wsem1):
    wid = lax.axis_index("s") * NUM_CORES + lax.axis_index("c")
    base = pl.multiple_of(wid * ROWS_PER_WORKER, ROWS_PER_WORKER)

    pltpu.sync_copy(len_hbm.at[pl.ds(base, ROWS_PER_WORKER)], len_v)
    for c in range(CHUNKS_PER_WORKER):
        for g in range(CHUNK // 16):
            lv = len_v[pl.ds(c * CHUNK + g * 16, 16)]
            idx_v[c, pl.ds(g * 16, 16)] = _bucket(lv)

    bufs = (buf0, buf1)
    gsems = (gsem0, gsem1)
    wsems = (wsem0, wsem1)

    def gather(c):
        b = c % 2
        return pltpu.async_copy(tab_hbm.at[idx_v.at[c]], bufs[b], gsems[b])

    gathers = [gather(0), gather(1)]
    scatters = [None, None]
    for c in range(CHUNKS_PER_WORKER):
        b = c % 2
        gathers[b].wait()
        scatters[b] = pltpu.async_copy(
            bufs[b], out_hbm.at[pl.ds(base + c * CHUNK, CHUNK)], wsems[b])
        if c + 2 < CHUNKS_PER_WORKER:
            scatters[b].wait()
            gathers[b] = gather(c + 2)
    scatters[0].wait()
    scatters[1].wait()


def kernel(chunks_length, start_pos, genre_id, distance_emb, genre_emb):
    del start_pos
    gid = jnp.asarray(genre_id, jnp.int32)
    genre_row = jnp.take(genre_emb, gid[None], axis=0)
    combined = jnp.concatenate(
        [distance_emb, jnp.broadcast_to(genre_row, (4, EMB))], axis=1)

    mesh = plsc.VectorSubcoreMesh(
        core_axis_name="c", subcore_axis_name="s",
        num_cores=NUM_CORES, num_subcores=NUM_SUBCORES)
    run = pl.kernel(
        _encode_body,
        out_type=jax.ShapeDtypeStruct((ROWS, OUT_W), jnp.float32),
        mesh=mesh,
        compiler_params=pltpu.CompilerParams(needs_layout_passes=False),
        scratch_types=[
            pltpu.VMEM((ROWS_PER_WORKER,), jnp.int32),
            pltpu.VMEM((CHUNKS_PER_WORKER, CHUNK), jnp.int32),
            pltpu.VMEM((CHUNK, OUT_W), jnp.float32),
            pltpu.VMEM((CHUNK, OUT_W), jnp.float32),
            pltpu.SemaphoreType.DMA,
            pltpu.SemaphoreType.DMA,
            pltpu.SemaphoreType.DMA,
            pltpu.SemaphoreType.DMA,
        ],
    )
    return run(chunks_length, combined)


# speedup vs baseline: 6.6099x; 6.6099x over previous
import jax
import jax.numpy as jnp
from jax import lax
from jax.experimental import pallas as pl
from jax.experimental.pallas import tpu as pltpu
from jax.experimental.pallas import tpu_sc as plsc

EMB = 128
OUT_W = 2 * EMB
ROWS = 16384
NUM_CORES = 2
NUM_SUBCORES = 16
NUM_WORKERS = NUM_CORES * NUM_SUBCORES
ROWS_PER_WORKER = ROWS // NUM_WORKERS
CHUNK = 128
CHUNKS_PER_WORKER = ROWS_PER_WORKER // CHUNK
GROUPS_PER_CHUNK = CHUNK // 16


def _bucket(lv):
    return ((jnp.minimum(lv, 2) - 1)
            + (jnp.minimum(lv, 4) >> 2)
            + (jnp.minimum(lv, 8) >> 3))


def _encode_body(len_hbm, tab_hbm, out_hbm,
                 len_v, tab_v, buf0, buf1, wsem0, wsem1):
    wid = lax.axis_index("s") * NUM_CORES + lax.axis_index("c")
    base = pl.multiple_of(wid * ROWS_PER_WORKER, ROWS_PER_WORKER)

    pltpu.sync_copy(tab_hbm, tab_v)
    pltpu.sync_copy(len_hbm.at[pl.ds(base, ROWS_PER_WORKER)], len_v)

    iota256 = lax.iota(jnp.int32, 16) * OUT_W

    bufs = (buf0, buf1)
    wsems = (wsem0, wsem1)
    pending = [None, None]
    for c in range(CHUNKS_PER_WORKER):
        b = c % 2
        if pending[b] is not None:
            pending[b].wait()
        buf = bufs[b]
        for g in range(GROUPS_PER_CHUNK):
            lv = len_v[pl.ds((c * GROUPS_PER_CHUNK + g) * 16, 16)]
            src0 = _bucket(lv) * OUT_W
            dst0 = iota256 + g * 16 * OUT_W

            @plsc.parallel_loop(0, OUT_W, unroll=8)
            def _(col):
                v = plsc.load_gather(tab_v, [src0 + col])
                plsc.store_scatter(buf, [dst0 + col], v)

        pending[b] = pltpu.async_copy(
            buf, out_hbm.at[pl.ds(base * OUT_W + c * CHUNK * OUT_W,
                                  CHUNK * OUT_W)], wsems[b])
    pending[0].wait()
    pending[1].wait()


def kernel(chunks_length, start_pos, genre_id, distance_emb, genre_emb):
    del start_pos
    gid = jnp.asarray(genre_id, jnp.int32)
    genre_row = jnp.take(genre_emb, gid[None], axis=0)
    combined = jnp.concatenate(
        [distance_emb, jnp.broadcast_to(genre_row, (4, EMB))], axis=1)

    mesh = plsc.VectorSubcoreMesh(
        core_axis_name="c", subcore_axis_name="s",
        num_cores=NUM_CORES, num_subcores=NUM_SUBCORES)
    run = pl.kernel(
        _encode_body,
        out_type=jax.ShapeDtypeStruct((ROWS * OUT_W,), jnp.float32),
        mesh=mesh,
        compiler_params=pltpu.CompilerParams(needs_layout_passes=False),
        scratch_types=[
            pltpu.VMEM((ROWS_PER_WORKER,), jnp.int32),
            pltpu.VMEM((4 * OUT_W,), jnp.float32),
            pltpu.VMEM((CHUNK * OUT_W,), jnp.float32),
            pltpu.VMEM((CHUNK * OUT_W,), jnp.float32),
            pltpu.SemaphoreType.DMA,
            pltpu.SemaphoreType.DMA,
        ],
    )
    return run(chunks_length, combined.reshape(-1)).reshape(ROWS, OUT_W)


# speedup vs baseline: 12.3149x; 1.8631x over previous
import jax
import jax.numpy as jnp
from jax import lax
from jax.experimental import pallas as pl
from jax.experimental.pallas import tpu as pltpu
from jax.experimental.pallas import tpu_sc as plsc

EMB = 128
OUT_W = 2 * EMB
ROWS = 16384
NUM_CORES = 2
NUM_SUBCORES = 16
NUM_WORKERS = NUM_CORES * NUM_SUBCORES
ROWS_PER_WORKER = ROWS // NUM_WORKERS
CHUNK = 128
CHUNKS_PER_WORKER = ROWS_PER_WORKER // CHUNK
GROUPS_PER_WORKER = ROWS_PER_WORKER // 16


def _bucket(lv):
    return ((jnp.minimum(lv, 2) - 1)
            + (jnp.minimum(lv, 4) >> 2)
            + (jnp.minimum(lv, 8) >> 3))


def _encode_body(len_hbm, tab_hbm, out_hbm,
                 len_v, tab_v, boff_v, buf0, buf1, wsem0, wsem1):
    wid = lax.axis_index("s") * NUM_CORES + lax.axis_index("c")
    base = pl.multiple_of(wid * ROWS_PER_WORKER, ROWS_PER_WORKER)

    pltpu.sync_copy(tab_hbm, tab_v)
    pltpu.sync_copy(len_hbm.at[pl.ds(base, ROWS_PER_WORKER)], len_v)

    for g in range(GROUPS_PER_WORKER):
        lv = len_v[pl.ds(g * 16, 16)]
        boff_v[pl.ds(g * 16, 16)] = _bucket(lv) * OUT_W

    iota16 = lax.iota(jnp.int32, 16)

    bufs = (buf0, buf1)
    wsems = (wsem0, wsem1)
    pending = [None, None]
    for c in range(CHUNKS_PER_WORKER):
        b = c % 2
        if pending[b] is not None:
            pending[b].wait()
        buf = bufs[b]

        @plsc.parallel_loop(0, CHUNK, unroll=2)
        def _(r):
            src0 = plsc.load_gather(boff_v, [jnp.broadcast_to(c * CHUNK + r, (16,))])
            src0 = src0 + iota16
            dst0 = jnp.broadcast_to(r * OUT_W, (16,)) + iota16
            for k in range(OUT_W // 16):
                v = plsc.load_gather(tab_v, [src0 + k * 16])
                plsc.store_scatter(buf, [dst0 + k * 16], v)

        pending[b] = pltpu.async_copy(
            buf, out_hbm.at[pl.ds(base * OUT_W + c * CHUNK * OUT_W,
                                  CHUNK * OUT_W)], wsems[b])
    pending[0].wait()
    pending[1].wait()


def kernel(chunks_length, start_pos, genre_id, distance_emb, genre_emb):
    del start_pos
    gid = jnp.asarray(genre_id, jnp.int32)
    genre_row = jnp.take(genre_emb, gid[None], axis=0)
    combined = jnp.concatenate(
        [distance_emb, jnp.broadcast_to(genre_row, (4, EMB))], axis=1)

    mesh = plsc.VectorSubcoreMesh(
        core_axis_name="c", subcore_axis_name="s",
        num_cores=NUM_CORES, num_subcores=NUM_SUBCORES)
    run = pl.kernel(
        _encode_body,
        out_type=jax.ShapeDtypeStruct((ROWS * OUT_W,), jnp.float32),
        mesh=mesh,
        compiler_params=pltpu.CompilerParams(needs_layout_passes=False),
        scratch_types=[
            pltpu.VMEM((ROWS_PER_WORKER,), jnp.int32),
            pltpu.VMEM((4 * OUT_W,), jnp.float32),
            pltpu.VMEM((ROWS_PER_WORKER,), jnp.int32),
            pltpu.VMEM((CHUNK * OUT_W,), jnp.float32),
            pltpu.VMEM((CHUNK * OUT_W,), jnp.float32),
            pltpu.SemaphoreType.DMA,
            pltpu.SemaphoreType.DMA,
        ],
    )
    return run(chunks_length, combined.reshape(-1)).reshape(ROWS, OUT_W)


# speedup vs baseline: 22.1213x; 1.7963x over previous
import jax
import jax.numpy as jnp
from jax import lax
from jax.experimental import pallas as pl
from jax.experimental.pallas import tpu as pltpu
from jax.experimental.pallas import tpu_sc as plsc

EMB = 128
OUT_W = 2 * EMB
ROWS = 16384
NUM_CORES = 2
NUM_SUBCORES = 16
NUM_WORKERS = NUM_CORES * NUM_SUBCORES
ROWS_PER_WORKER = ROWS // NUM_WORKERS
CHUNK = 128
CHUNKS_PER_WORKER = ROWS_PER_WORKER // CHUNK
GROUPS_PER_WORKER = ROWS_PER_WORKER // 16


def _bucket(lv):
    return ((jnp.minimum(lv, 2) - 1)
            + (jnp.minimum(lv, 4) >> 2)
            + (jnp.minimum(lv, 8) >> 3))


def _encode_body(len_hbm, tab_hbm, out_hbm,
                 len_v, tab_v, boff_v, buf0, buf1, wsem0, wsem1):
    wid = lax.axis_index("s") * NUM_CORES + lax.axis_index("c")
    base = pl.multiple_of(wid * ROWS_PER_WORKER, ROWS_PER_WORKER)

    pltpu.sync_copy(tab_hbm, tab_v)
    pltpu.sync_copy(len_hbm.at[pl.ds(base, ROWS_PER_WORKER)], len_v)

    for g in range(GROUPS_PER_WORKER):
        lv = len_v[pl.ds(g * 16, 16)]
        boff_v[pl.ds(g * 16, 16)] = _bucket(lv) * OUT_W

    iota16 = lax.iota(jnp.int32, 16)

    bufs = (buf0, buf1)
    wsems = (wsem0, wsem1)
    pending = [None, None]
    for c in range(CHUNKS_PER_WORKER):
        b = c % 2
        if pending[b] is not None:
            pending[b].wait()
        buf = bufs[b]

        @plsc.parallel_loop(0, CHUNK, unroll=2)
        def _(r):
            src0 = plsc.load_gather(boff_v, [jnp.broadcast_to(c * CHUNK + r, (16,))])
            src0 = src0 + iota16
            row = jnp.broadcast_to(r, (16,))
            for k in range(OUT_W // 16):
                v = plsc.load_gather(tab_v, [src0 + k * 16])
                plsc.store_scatter(buf, [row, iota16 + k * 16], v)

        pending[b] = pltpu.async_copy(
            buf, out_hbm.at[pl.ds(base + c * CHUNK, CHUNK)], wsems[b])
    pending[0].wait()
    pending[1].wait()


def kernel(chunks_length, start_pos, genre_id, distance_emb, genre_emb):
    del start_pos
    gid = jnp.asarray(genre_id, jnp.int32)
    genre_row = jnp.take(genre_emb, gid[None], axis=0)
    combined = jnp.concatenate(
        [distance_emb, jnp.broadcast_to(genre_row, (4, EMB))], axis=1)

    mesh = plsc.VectorSubcoreMesh(
        core_axis_name="c", subcore_axis_name="s",
        num_cores=NUM_CORES, num_subcores=NUM_SUBCORES)
    run = pl.kernel(
        _encode_body,
        out_type=jax.ShapeDtypeStruct((ROWS, OUT_W), jnp.float32),
        mesh=mesh,
        compiler_params=pltpu.CompilerParams(needs_layout_passes=False),
        scratch_types=[
            pltpu.VMEM((ROWS_PER_WORKER,), jnp.int32),
            pltpu.VMEM((4 * OUT_W,), jnp.float32),
            pltpu.VMEM((ROWS_PER_WORKER,), jnp.int32),
            pltpu.VMEM((CHUNK, OUT_W), jnp.float32),
            pltpu.VMEM((CHUNK, OUT_W), jnp.float32),
            pltpu.SemaphoreType.DMA,
            pltpu.SemaphoreType.DMA,
        ],
    )
    return run(chunks_length, combined.reshape(-1))
